# baseline (speedup 1.0000x reference)
import functools

import jax
import jax.numpy as jnp
from jax import lax
from jax.experimental import pallas as pl
from jax.experimental.pallas import tpu as pltpu
from jax.experimental.pallas import tpu_sc as plsc

NC = 2
NS = 16
NW = NC * NS
B = 16384
D = 64
BPW = B // NW
GB = 128
NCH = BPW // GB
CW = 6400
KU = 256000
KI = 25600
H = D // 2


def _relayout_body(a_ref, b_ref, c_ref, d_ref, o_ref):
    eye = (lax.broadcasted_iota(jnp.int32, (D, D), 0) ==
           lax.broadcasted_iota(jnp.int32, (D, D), 1)).astype(jnp.float32)

    def quad(ref):
        rows = lax.dot_general(ref[...], eye, (((0,), (0,)), ((), ())),
                               preferred_element_type=jnp.float32)
        b = rows.astype(jnp.bfloat16)
        lo = lax.bitcast_convert_type(b[:, :H], jnp.uint16).astype(jnp.uint32)
        hi = lax.bitcast_convert_type(b[:, H:], jnp.uint16).astype(jnp.uint32)
        return lax.bitcast_convert_type(lo | (hi << 16), jnp.int32)

    o_ref[...] = jnp.concatenate(
        [quad(a_ref), quad(b_ref), quad(c_ref), quad(d_ref)], axis=1)


def _relayout(tabT, k):
    nb = k // CW
    last = (tabT.shape[1] - 1) // CW
    return pl.pallas_call(
        _relayout_body,
        grid=(nb,),
        in_specs=[
            pl.BlockSpec((D, CW), lambda m: (0, m)),
            pl.BlockSpec((D, CW), lambda m: (0, jnp.minimum(m + nb, last))),
            pl.BlockSpec((D, CW),
                         lambda m: (0, jnp.minimum(m + 2 * nb, last))),
            pl.BlockSpec((D, CW),
                         lambda m: (0, jnp.minimum(m + 3 * nb, last))),
        ],
        out_specs=pl.BlockSpec((CW, 2 * D), lambda m: (m, 0)),
        out_shape=jax.ShapeDtypeStruct((k, 2 * D), jnp.int32),
    )(tabT, tabT, tabT, tabT)


def _gather_body(users_hbm, isbns_hbm, utab_hbm, itab_hbm,
                 uout_hbm, iout_hbm,
                 uidx_v, iidx_v, ugi_v, igi_v, pair_v, usem, isem):
    wid = lax.axis_index("s") * NC + lax.axis_index("c")
    base = wid * BPW
    pltpu.sync_copy(users_hbm.at[pl.ds(base, BPW)], uidx_v)
    pltpu.sync_copy(isbns_hbm.at[pl.ds(base, BPW)], iidx_v)
    for k in range(BPW // 16):
        c, l = divmod(k, 8)
        u = uidx_v[pl.ds(k * 16, 16)]
        u = jnp.where(u >= 2 * KU, u - 2 * KU, u)
        ugi_v[c, pl.ds(l * 16, 16)] = jnp.where(u >= KU, u - KU, u)
        i = iidx_v[pl.ds(k * 16, 16)]
        i = jnp.where(i >= 2 * KI, i - 2 * KI, i)
        igi_v[c, pl.ds(l * 16, 16)] = jnp.where(i >= KI, i - KI, i)

    ucopies = [pltpu.async_copy(utab_hbm.at[ugi_v.at[c]],
                                pair_v.at[c], usem)
               for c in range(NCH)]
    for cp in ucopies:
        cp.wait()
    pltpu.sync_copy(pair_v, uout_hbm.at[wid])
    icopies = [pltpu.async_copy(itab_hbm.at[igi_v.at[c]],
                                pair_v.at[c], isem)
               for c in range(NCH)]
    for cp in icopies:
        cp.wait()
    pltpu.sync_copy(pair_v, iout_hbm.at[wid])


def _sc_gather(users, isbns, upacked, ipacked):
    mesh = plsc.VectorSubcoreMesh(core_axis_name="c", subcore_axis_name="s")
    f = pl.kernel(
        _gather_body,
        out_type=(
            jax.ShapeDtypeStruct((NW, NCH, GB, 2 * D), jnp.int32),
            jax.ShapeDtypeStruct((NW, NCH, GB, 2 * D), jnp.int32),
        ),
        mesh=mesh,
        scratch_types=[
            pltpu.VMEM((BPW,), jnp.int32),
            pltpu.VMEM((BPW,), jnp.int32),
            pltpu.VMEM((NCH, GB), jnp.int32),
            pltpu.VMEM((NCH, GB), jnp.int32),
            pltpu.VMEM((NCH, GB, 2 * D), jnp.int32),
            pltpu.SemaphoreType.DMA,
            pltpu.SemaphoreType.DMA,
        ],
    )
    return f(users, isbns, upacked, ipacked)


BM = 2048


def _unpack_select(p_ref, q_ref):
    p = p_ref[...]
    q = q_ref[...]
    sel = jnp.where(q > 1.5,
                    jnp.where(q > 2.5, p[:, 3 * H:4 * H], p[:, 2 * H:3 * H]),
                    jnp.where(q > 0.5, p[:, H:2 * H], p[:, :H]))
    lo = lax.bitcast_convert_type(sel << 16, jnp.float32)
    hi = lax.bitcast_convert_type(sel & jnp.int32(-65536), jnp.float32)
    return jnp.concatenate([lo, hi], axis=1)


def _mlp_body(up_ref, ip_ref, uhi_ref, ihi_ref, w1u_ref, w1i_ref, b1_ref,
              w2_ref, b2_ref, w3_ref, b3_ref, o_ref):
    xu = _unpack_select(up_ref, uhi_ref)
    xi = _unpack_select(ip_ref, ihi_ref)
    x = jnp.dot(xu, w1u_ref[...], preferred_element_type=jnp.float32)
    x = x + jnp.dot(xi, w1i_ref[...], preferred_element_type=jnp.float32)
    x = jnp.maximum(x + b1_ref[...], 0.0)
    x = jnp.maximum(
        jnp.dot(x, w2_ref[...], preferred_element_type=jnp.float32)
        + b2_ref[...], 0.0)
    o_ref[...] = (jnp.dot(x, w3_ref[...], preferred_element_type=jnp.float32)
                  + b3_ref[...])


def _tc_mlp(upairs, ipairs, uhi, ihi, W1, b1, W2, b2, W3, b3):
    full = lambda s: pl.BlockSpec(s, lambda m: (0, 0))
    return pl.pallas_call(
        _mlp_body,
        grid=(B // BM,),
        in_specs=[
            pl.BlockSpec((BM, 2 * D), lambda m: (m, 0)),
            pl.BlockSpec((BM, 2 * D), lambda m: (m, 0)),
            pl.BlockSpec((BM, 1), lambda m: (m, 0)),
            pl.BlockSpec((BM, 1), lambda m: (m, 0)),
            full((D, 64)),
            full((D, 64)),
            full((1, 64)),
            full((64, 32)),
            full((1, 32)),
            full((32, 1)),
            full((1, 1)),
        ],
        out_specs=pl.BlockSpec((BM, 1), lambda m: (m, 0)),
        out_shape=jax.ShapeDtypeStruct((B, 1), jnp.float32),
    )(upairs, ipairs, uhi, ihi,
      W1[:D], W1[D:], b1.reshape(1, 64), W2, b2.reshape(1, 32),
      W3, b3.reshape(1, 1))


def kernel(users, isbns, user_table, isbn_table, W1, b1, W2, b2, W3, b3):
    upacked = _relayout(user_table.T, KU)
    ipacked = _relayout(isbn_table.T, KI)
    upairs, ipairs = _sc_gather(users, isbns, upacked, ipacked)
    upairs = upairs.reshape(B, 2 * D)
    ipairs = ipairs.reshape(B, 2 * D)
    uhi = (users // KU).astype(jnp.float32).reshape(B, 1)
    ihi = (isbns // KI).astype(jnp.float32).reshape(B, 1)
    return _tc_mlp(upairs, ipairs, uhi, ihi, W1, b1, W2, b2, W3, b3)


# speedup vs baseline: 1.7994x; 1.7994x over previous
import functools

import jax
import jax.numpy as jnp
from jax import lax
from jax.experimental import pallas as pl
from jax.experimental.pallas import tpu as pltpu
from jax.experimental.pallas import tpu_sc as plsc

NC = 2
NS = 16
NW = NC * NS
B = 16384
D = 64
BPW = B // NW
GB = 128
NCH = BPW // GB
CW = 6400
KU = 256000
KI = 25600
H = D // 2


def _relayout_body(a_ref, b_ref, c_ref, d_ref, o_ref):
    eye = (lax.broadcasted_iota(jnp.int32, (D, D), 0) ==
           lax.broadcasted_iota(jnp.int32, (D, D), 1)).astype(jnp.bfloat16)

    def rows_of(ref):
        rows = lax.dot_general(ref[...].astype(jnp.bfloat16), eye,
                               (((0,), (0,)), ((), ())),
                               preferred_element_type=jnp.float32)
        return lax.bitcast_convert_type(rows, jnp.int32)

    def pack(lo_bits, hi_bits):
        return ((hi_bits & jnp.int32(-65536)) |
                lax.shift_right_logical(lo_bits, 16))

    a, b, c, d = (rows_of(r) for r in (a_ref, b_ref, c_ref, d_ref))
    o_ref[...] = jnp.concatenate([pack(a, c), pack(b, d)], axis=1)


def _relayout(tabT, k):
    nb = k // CW
    last = (tabT.shape[1] - 1) // CW
    return pl.pallas_call(
        _relayout_body,
        grid=(nb,),
        in_specs=[
            pl.BlockSpec((D, CW), lambda m: (0, m)),
            pl.BlockSpec((D, CW), lambda m: (0, jnp.minimum(m + nb, last))),
            pl.BlockSpec((D, CW),
                         lambda m: (0, jnp.minimum(m + 2 * nb, last))),
            pl.BlockSpec((D, CW),
                         lambda m: (0, jnp.minimum(m + 3 * nb, last))),
        ],
        out_specs=pl.BlockSpec((CW, 2 * D), lambda m: (m, 0)),
        out_shape=jax.ShapeDtypeStruct((k, 2 * D), jnp.int32),
    )(tabT, tabT, tabT, tabT)


def _gather_body(users_hbm, isbns_hbm, utab_hbm, itab_hbm,
                 uout_hbm, iout_hbm,
                 uidx_v, iidx_v, ugi_v, igi_v, pair_v, usem, isem):
    wid = lax.axis_index("s") * NC + lax.axis_index("c")
    base = wid * BPW
    pltpu.sync_copy(users_hbm.at[pl.ds(base, BPW)], uidx_v)
    pltpu.sync_copy(isbns_hbm.at[pl.ds(base, BPW)], iidx_v)
    for k in range(BPW // 16):
        c, l = divmod(k, 8)
        u = uidx_v[pl.ds(k * 16, 16)]
        u = jnp.where(u >= 2 * KU, u - 2 * KU, u)
        ugi_v[c, pl.ds(l * 16, 16)] = jnp.where(u >= KU, u - KU, u)
        i = iidx_v[pl.ds(k * 16, 16)]
        i = jnp.where(i >= 2 * KI, i - 2 * KI, i)
        igi_v[c, pl.ds(l * 16, 16)] = jnp.where(i >= KI, i - KI, i)

    ucopies = [pltpu.async_copy(utab_hbm.at[ugi_v.at[c]],
                                pair_v.at[c], usem)
               for c in range(NCH)]
    for cp in ucopies:
        cp.wait()
    pltpu.sync_copy(pair_v, uout_hbm.at[wid])
    icopies = [pltpu.async_copy(itab_hbm.at[igi_v.at[c]],
                                pair_v.at[c], isem)
               for c in range(NCH)]
    for cp in icopies:
        cp.wait()
    pltpu.sync_copy(pair_v, iout_hbm.at[wid])


def _sc_gather(users, isbns, upacked, ipacked):
    mesh = plsc.VectorSubcoreMesh(core_axis_name="c", subcore_axis_name="s")
    f = pl.kernel(
        _gather_body,
        out_type=(
            jax.ShapeDtypeStruct((NW, NCH, GB, 2 * D), jnp.int32),
            jax.ShapeDtypeStruct((NW, NCH, GB, 2 * D), jnp.int32),
        ),
        mesh=mesh,
        scratch_types=[
            pltpu.VMEM((BPW,), jnp.int32),
            pltpu.VMEM((BPW,), jnp.int32),
            pltpu.VMEM((NCH, GB), jnp.int32),
            pltpu.VMEM((NCH, GB), jnp.int32),
            pltpu.VMEM((NCH, GB, 2 * D), jnp.int32),
            pltpu.SemaphoreType.DMA,
            pltpu.SemaphoreType.DMA,
        ],
    )
    return f(users, isbns, upacked, ipacked)


BM = 2048


def _unpack_select(p_ref, odd_ref, low_ref):
    p = p_ref[...]
    sel = jnp.where(odd_ref[...] > 0, p[:, D:], p[:, :D])
    bits = jnp.where(low_ref[...] > 0, sel << 16, sel & jnp.int32(-65536))
    return lax.bitcast_convert_type(bits, jnp.float32)


def _mlp_body(up_ref, ip_ref, uo_ref, ul_ref, io_ref, il_ref,
              w1u_ref, w1i_ref, b1_ref,
              w2_ref, b2_ref, w3_ref, b3_ref, o_ref):
    xu = _unpack_select(up_ref, uo_ref, ul_ref)
    xi = _unpack_select(ip_ref, io_ref, il_ref)
    x = jnp.dot(xu, w1u_ref[...], preferred_element_type=jnp.float32)
    x = x + jnp.dot(xi, w1i_ref[...], preferred_element_type=jnp.float32)
    x = jnp.maximum(x + b1_ref[...], 0.0)
    x = jnp.maximum(
        jnp.dot(x, w2_ref[...], preferred_element_type=jnp.float32)
        + b2_ref[...], 0.0)
    o_ref[...] = (jnp.dot(x, w3_ref[...], preferred_element_type=jnp.float32)
                  + b3_ref[...])


def _tc_mlp(upairs, ipairs, uflags, iflags, W1, b1, W2, b2, W3, b3):
    full = lambda s: pl.BlockSpec(s, lambda m: (0, 0))
    return pl.pallas_call(
        _mlp_body,
        grid=(B // BM,),
        in_specs=[
            pl.BlockSpec((BM, 2 * D), lambda m: (m, 0)),
            pl.BlockSpec((BM, 2 * D), lambda m: (m, 0)),
            pl.BlockSpec((BM, 1), lambda m: (m, 0)),
            pl.BlockSpec((BM, 1), lambda m: (m, 0)),
            pl.BlockSpec((BM, 1), lambda m: (m, 0)),
            pl.BlockSpec((BM, 1), lambda m: (m, 0)),
            full((D, 64)),
            full((D, 64)),
            full((1, 64)),
            full((64, 32)),
            full((1, 32)),
            full((32, 1)),
            full((1, 1)),
        ],
        out_specs=pl.BlockSpec((BM, 1), lambda m: (m, 0)),
        out_shape=jax.ShapeDtypeStruct((B, 1), jnp.float32),
    )(upairs, ipairs, *uflags, *iflags,
      W1[:D], W1[D:], b1.reshape(1, 64), W2, b2.reshape(1, 32),
      W3, b3.reshape(1, 1))


def kernel(users, isbns, user_table, isbn_table, W1, b1, W2, b2, W3, b3):
    upacked = _relayout(user_table.T, KU)
    ipacked = _relayout(isbn_table.T, KI)
    upairs, ipairs = _sc_gather(users, isbns, upacked, ipacked)
    upairs = upairs.reshape(B, 2 * D)
    ipairs = ipairs.reshape(B, 2 * D)
    uq = users // KU
    iq = isbns // KI
    uflags = ((uq & 1).astype(jnp.float32).reshape(B, 1),
              (uq < 2).astype(jnp.float32).reshape(B, 1))
    iflags = ((iq & 1).astype(jnp.float32).reshape(B, 1),
              (iq < 2).astype(jnp.float32).reshape(B, 1))
    return _tc_mlp(upairs, ipairs, uflags, iflags, W1, b1, W2, b2, W3, b3)


# speedup vs baseline: 1.8766x; 1.0429x over previous
import functools

import jax
import jax.numpy as jnp
from jax import lax
from jax.experimental import pallas as pl
from jax.experimental.pallas import tpu as pltpu
from jax.experimental.pallas import tpu_sc as plsc

NC = 2
NS = 16
NW = NC * NS
B = 16384
D = 64
BPW = B // NW
GB = 128
NCH = BPW // GB
CW = 12800
KU = 256000
KI = 25600
H = D // 2


def _relayout_body(a_ref, b_ref, c_ref, d_ref, o_ref):
    eye = (lax.broadcasted_iota(jnp.int32, (D, D), 0) ==
           lax.broadcasted_iota(jnp.int32, (D, D), 1)).astype(jnp.bfloat16)

    def rows_of(ref):
        rows = lax.dot_general(ref[...].astype(jnp.bfloat16), eye,
                               (((0,), (0,)), ((), ())),
                               preferred_element_type=jnp.float32)
        return lax.bitcast_convert_type(rows, jnp.int32)

    def pack(lo_bits, hi_bits):
        return ((hi_bits & jnp.int32(-65536)) |
                lax.shift_right_logical(lo_bits, 16))

    a, b, c, d = (rows_of(r) for r in (a_ref, b_ref, c_ref, d_ref))
    o_ref[...] = jnp.concatenate([pack(a, c), pack(b, d)], axis=1)


def _relayout(tabT, k):
    nb = k // CW
    last = (tabT.shape[1] - 1) // CW
    return pl.pallas_call(
        _relayout_body,
        grid=(nb,),
        in_specs=[
            pl.BlockSpec((D, CW), lambda m: (0, m)),
            pl.BlockSpec((D, CW), lambda m: (0, jnp.minimum(m + nb, last))),
            pl.BlockSpec((D, CW),
                         lambda m: (0, jnp.minimum(m + 2 * nb, last))),
            pl.BlockSpec((D, CW),
                         lambda m: (0, jnp.minimum(m + 3 * nb, last))),
        ],
        out_specs=pl.BlockSpec((CW, 2 * D), lambda m: (m, 0)),
        out_shape=jax.ShapeDtypeStruct((k, 2 * D), jnp.int32),
    )(tabT, tabT, tabT, tabT)


def _gather_body(users_hbm, isbns_hbm, utab_hbm, itab_hbm,
                 uout_hbm, iout_hbm,
                 uidx_v, iidx_v, ugi_v, igi_v, pair_v, usem, isem):
    wid = lax.axis_index("s") * NC + lax.axis_index("c")
    base = wid * BPW
    pltpu.sync_copy(users_hbm.at[pl.ds(base, BPW)], uidx_v)
    pltpu.sync_copy(isbns_hbm.at[pl.ds(base, BPW)], iidx_v)
    for k in range(BPW // 16):
        c, l = divmod(k, 8)
        u = uidx_v[pl.ds(k * 16, 16)]
        u = jnp.where(u >= 2 * KU, u - 2 * KU, u)
        ugi_v[c, pl.ds(l * 16, 16)] = jnp.where(u >= KU, u - KU, u)
        i = iidx_v[pl.ds(k * 16, 16)]
        i = jnp.where(i >= 2 * KI, i - 2 * KI, i)
        igi_v[c, pl.ds(l * 16, 16)] = jnp.where(i >= KI, i - KI, i)

    ucopies = [pltpu.async_copy(utab_hbm.at[ugi_v.at[c]],
                                pair_v.at[c], usem)
               for c in range(NCH)]
    for cp in ucopies:
        cp.wait()
    pltpu.sync_copy(pair_v, uout_hbm.at[wid])
    icopies = [pltpu.async_copy(itab_hbm.at[igi_v.at[c]],
                                pair_v.at[c], isem)
               for c in range(NCH)]
    for cp in icopies:
        cp.wait()
    pltpu.sync_copy(pair_v, iout_hbm.at[wid])


def _sc_gather(users, isbns, upacked, ipacked):
    mesh = plsc.VectorSubcoreMesh(core_axis_name="c", subcore_axis_name="s")
    f = pl.kernel(
        _gather_body,
        out_type=(
            jax.ShapeDtypeStruct((NW, NCH, GB, 2 * D), jnp.int32),
            jax.ShapeDtypeStruct((NW, NCH, GB, 2 * D), jnp.int32),
        ),
        mesh=mesh,
        scratch_types=[
            pltpu.VMEM((BPW,), jnp.int32),
            pltpu.VMEM((BPW,), jnp.int32),
            pltpu.VMEM((NCH, GB), jnp.int32),
            pltpu.VMEM((NCH, GB), jnp.int32),
            pltpu.VMEM((NCH, GB, 2 * D), jnp.int32),
            pltpu.SemaphoreType.DMA,
            pltpu.SemaphoreType.DMA,
        ],
    )
    return f(users, isbns, upacked, ipacked)


BM = 2048


def _unpack_select(p_ref, odd_ref, low_ref):
    p = p_ref[...]
    sel = jnp.where(odd_ref[...] > 0, p[:, D:], p[:, :D])
    bits = jnp.where(low_ref[...] > 0, sel << 16, sel & jnp.int32(-65536))
    return lax.bitcast_convert_type(bits, jnp.float32)


def _mlp_body(up_ref, ip_ref, uo_ref, ul_ref, io_ref, il_ref,
              w1u_ref, w1i_ref, b1_ref,
              w2_ref, b2_ref, w3_ref, b3_ref, o_ref):
    xu = _unpack_select(up_ref, uo_ref, ul_ref)
    xi = _unpack_select(ip_ref, io_ref, il_ref)
    x = jnp.dot(xu, w1u_ref[...], preferred_element_type=jnp.float32)
    x = x + jnp.dot(xi, w1i_ref[...], preferred_element_type=jnp.float32)
    x = jnp.maximum(x + b1_ref[...], 0.0)
    x = jnp.maximum(
        jnp.dot(x, w2_ref[...], preferred_element_type=jnp.float32)
        + b2_ref[...], 0.0)
    o_ref[...] = (jnp.dot(x, w3_ref[...], preferred_element_type=jnp.float32)
                  + b3_ref[...])


def _tc_mlp(upairs, ipairs, uflags, iflags, W1, b1, W2, b2, W3, b3):
    full = lambda s: pl.BlockSpec(s, lambda m: (0, 0))
    return pl.pallas_call(
        _mlp_body,
        grid=(B // BM,),
        in_specs=[
            pl.BlockSpec((BM, 2 * D), lambda m: (m, 0)),
            pl.BlockSpec((BM, 2 * D), lambda m: (m, 0)),
            pl.BlockSpec((BM, 1), lambda m: (m, 0)),
            pl.BlockSpec((BM, 1), lambda m: (m, 0)),
            pl.BlockSpec((BM, 1), lambda m: (m, 0)),
            pl.BlockSpec((BM, 1), lambda m: (m, 0)),
            full((D, 64)),
            full((D, 64)),
            full((1, 64)),
            full((64, 32)),
            full((1, 32)),
            full((32, 1)),
            full((1, 1)),
        ],
        out_specs=pl.BlockSpec((BM, 1), lambda m: (m, 0)),
        out_shape=jax.ShapeDtypeStruct((B, 1), jnp.float32),
    )(upairs, ipairs, *uflags, *iflags,
      W1[:D], W1[D:], b1.reshape(1, 64), W2, b2.reshape(1, 32),
      W3, b3.reshape(1, 1))


def kernel(users, isbns, user_table, isbn_table, W1, b1, W2, b2, W3, b3):
    upacked = _relayout(user_table.T, KU)
    ipacked = _relayout(isbn_table.T, KI)
    upairs, ipairs = _sc_gather(users, isbns, upacked, ipacked)
    upairs = upairs.reshape(B, 2 * D)
    ipairs = ipairs.reshape(B, 2 * D)
    uq = users // KU
    iq = isbns // KI
    uflags = ((uq & 1).astype(jnp.float32).reshape(B, 1),
              (uq < 2).astype(jnp.float32).reshape(B, 1))
    iflags = ((iq & 1).astype(jnp.float32).reshape(B, 1),
              (iq < 2).astype(jnp.float32).reshape(B, 1))
    return _tc_mlp(upairs, ipairs, uflags, iflags, W1, b1, W2, b2, W3, b3)
